# baseline (speedup 1.0000x reference)
import functools

import jax
import jax.numpy as jnp
from jax import lax
from jax.experimental import pallas as pl
from jax.experimental.pallas import tpu as pltpu
from jax.experimental.pallas import tpu_sc as plsc

F32 = jnp.float32

NC = 2
NS = 16
LANES = 16
GK = 4
GROUP = GK * 128
DGROUP = 1024


def _sc_deg_call(dstp, nslab, epad, mesh):
    nw = NC * NS
    epw = epad // nw

    def body(dst_ref, out_ref, didx, hist):
        c = lax.axis_index("c")
        s = lax.axis_index("s")
        w = s * NC + c
        z16 = jnp.zeros((LANES,), F32)

        def zb(i, _):
            hist[pl.ds(i * LANES, LANES)] = z16
            return 0

        lax.fori_loop(0, nslab // LANES, zb, 0)
        ones16 = jnp.ones((LANES,), F32)

        def gb(g, _):
            e0 = w * epw + g * DGROUP
            pltpu.sync_copy(dst_ref.at[pl.ds(e0, DGROUP)], didx)

            def ib(i, _):
                v = didx[pl.ds(i * LANES, LANES)]
                plsc.addupdate_scatter(hist, [v], ones16)
                return 0

            lax.fori_loop(0, DGROUP // LANES, ib, 0)
            return 0

        lax.fori_loop(0, epw // DGROUP, gb, 0)
        pltpu.sync_copy(hist, out_ref.at[w])

    f = pl.kernel(
        body,
        out_type=jax.ShapeDtypeStruct((nw, nslab), F32),
        mesh=mesh,
        scratch_types=[
            pltpu.VMEM((DGROUP,), jnp.int32),
            pltpu.VMEM((nslab,), F32),
        ],
        compiler_params=pltpu.CompilerParams(needs_layout_passes=False),
    )
    return f(dstp)


def _sc_layer_call(h2, gir, dstr, zeros, nslab, ept, mesh):
    rpt = ept // 128
    zpt = nslab // NS
    ngroups = ept // GROUP

    def body(h2_ref, gi_ref, dst_ref, z_ref, out_ref,
             gidx, didx, rows, slab, gsem, ssem, isem):
        c = lax.axis_index("c")
        s = lax.axis_index("s")
        pltpu.sync_copy(z_ref.at[pl.ds(s * zpt, zpt)],
                        slab.at[pl.ds(s * zpt, zpt)])
        plsc.subcore_barrier()

        def prefetch_idx(g, p):
            e0 = s * ept + g * GROUP
            pltpu.async_copy(gi_ref.at[c, pl.ds(e0, GROUP)], gidx.at[p], isem)
            pltpu.async_copy(dst_ref.at[pl.ds(e0, GROUP)], didx.at[p], isem)

        def wait_idx(g, p):
            e0 = s * ept + g * GROUP
            pltpu.make_async_copy(gi_ref.at[c, pl.ds(e0, GROUP)],
                                  gidx.at[p], isem).wait()
            pltpu.make_async_copy(dst_ref.at[pl.ds(e0, GROUP)],
                                  didx.at[p], isem).wait()

        def fire_gather(p):
            pltpu.async_copy(h2_ref.at[gidx.at[p]], rows.at[p], gsem)

        def wait_gather(p):
            pltpu.make_async_copy(h2_ref.at[gidx.at[p]], rows.at[p],
                                  gsem).wait()

        def fire_scatter(p):
            pltpu.async_copy(rows.at[p], slab.at[didx.at[p]], ssem, add=True)

        def drain_scatter(p):
            pltpu.make_async_copy(rows.at[p], slab.at[didx.at[p]],
                                  ssem).wait()

        prefetch_idx(0, 0)
        prefetch_idx(1, 1)
        wait_idx(0, 0)
        fire_gather(0)

        def gb(g, _):
            r = lax.rem(g, 3)
            r1 = lax.rem(g + 1, 3)
            r2 = lax.rem(g + 2, 3)

            @pl.when(g < ngroups - 1)
            def _():
                wait_idx(g + 1, r1)
                fire_gather(r1)

            @pl.when(g >= 1)
            def _():
                drain_scatter(r2)

            @pl.when(g < ngroups - 2)
            def _():
                prefetch_idx(g + 2, r2)

            wait_gather(r)
            fire_scatter(r)
            return 0

        lax.fori_loop(0, ngroups, gb, 0)
        drain_scatter(lax.rem(ngroups - 1, 3))
        plsc.subcore_barrier()
        pltpu.sync_copy(slab.at[pl.ds(s * zpt, zpt)],
                        out_ref.at[c, pl.ds(s * zpt, zpt)])

    f = pl.kernel(
        body,
        out_type=jax.ShapeDtypeStruct((NC, nslab, LANES), F32),
        mesh=mesh,
        scratch_types=[
            pltpu.VMEM((3, GROUP), jnp.int32),
            pltpu.VMEM((3, GROUP), jnp.int32),
            pltpu.VMEM((3, GROUP, LANES), F32),
            pltpu.VMEM_SHARED((nslab, LANES), F32),
            pltpu.SemaphoreType.DMA,
            pltpu.SemaphoreType.DMA,
            pltpu.SemaphoreType.DMA,
        ],
        compiler_params=pltpu.CompilerParams(use_tc_tiling_on_sc=False),
    )
    return f(h2, gir, dstr, zeros)


def _tc_proj(x, w1, b1, w2, b2, nblk):
    nsl, f_in = x.shape
    d = w2.shape[1]
    hid = w1.shape[1]

    def body(x_ref, w1_ref, b1_ref, w2_ref, b2_ref, h_ref):
        z = jnp.dot(x_ref[...], w1_ref[...], preferred_element_type=F32)
        z = jnp.maximum(z + b1_ref[...], 0.0)
        h_ref[...] = jnp.dot(z, w2_ref[...],
                             preferred_element_type=F32) + b2_ref[...]

    return pl.pallas_call(
        body,
        grid=(nsl // nblk,),
        in_specs=[
            pl.BlockSpec((nblk, f_in), lambda i: (i, 0)),
            pl.BlockSpec((f_in, hid), lambda i: (0, 0)),
            pl.BlockSpec((1, hid), lambda i: (0, 0)),
            pl.BlockSpec((hid, d), lambda i: (0, 0)),
            pl.BlockSpec((1, d), lambda i: (0, 0)),
        ],
        out_specs=pl.BlockSpec((nblk, d), lambda i: (i, 0)),
        out_shape=jax.ShapeDtypeStruct((nsl, d), F32),
    )(x, w1, b1, w2, b2)


def _tc_block(h, a0, a1, degp, w1h, w1a, w1b, b1, w2, b2,
              gam, bet, nblk):
    nsl, d = h.shape
    hid = w2.shape[0]
    nw = degp.shape[0]

    def body(h_ref, a0_ref, a1_ref, degp_ref, w1h_ref, w1a_ref, w1b_ref,
             b1_ref, w2_ref, b2_ref, gam_ref, bet_ref, o_ref):
        h_ = h_ref[...]
        deg = lax.dot_general(degp_ref[...], jnp.ones((nw, 1), F32),
                              (((0,), (0,)), ((), ())),
                              preferred_element_type=F32)
        di = 1.0 / jnp.maximum(deg, 1.0)
        z = (jnp.dot(h_, w1h_ref[...], preferred_element_type=F32)
             + jnp.dot(a0_ref[...] * di, w1a_ref[...],
                       preferred_element_type=F32)
             + jnp.dot(a1_ref[...] * di, w1b_ref[...],
                       preferred_element_type=F32)
             + b1_ref[...])
        z = jnp.maximum(z, 0.0)
        hn = jnp.dot(z, w2_ref[...], preferred_element_type=F32) + b2_ref[...]
        r = h_ + hn
        mu = jnp.mean(r, axis=1, keepdims=True)
        var = jnp.mean((r - mu) ** 2, axis=1, keepdims=True)
        o_ref[...] = ((r - mu) / jnp.sqrt(var + 1e-5) * gam_ref[...]
                      + bet_ref[...])

    return pl.pallas_call(
        body,
        grid=(nsl // nblk,),
        in_specs=[
            pl.BlockSpec((nblk, d), lambda i: (i, 0)),
            pl.BlockSpec((nblk, LANES), lambda i: (i, 0)),
            pl.BlockSpec((nblk, LANES), lambda i: (i, 0)),
            pl.BlockSpec((nw, nblk), lambda i: (0, i)),
            pl.BlockSpec((d, hid), lambda i: (0, 0)),
            pl.BlockSpec((LANES, hid), lambda i: (0, 0)),
            pl.BlockSpec((LANES, hid), lambda i: (0, 0)),
            pl.BlockSpec((1, hid), lambda i: (0, 0)),
            pl.BlockSpec((hid, d), lambda i: (0, 0)),
            pl.BlockSpec((1, d), lambda i: (0, 0)),
            pl.BlockSpec((1, d), lambda i: (0, 0)),
            pl.BlockSpec((1, d), lambda i: (0, 0)),
        ],
        out_specs=pl.BlockSpec((nblk, d), lambda i: (i, 0)),
        out_shape=jax.ShapeDtypeStruct((nsl, d), F32),
    )(h, a0, a1, degp, w1h, w1a, w1b, b1, w2, b2, gam, bet)


def _tc_out(h, w1, b1, w2, b2, nblk):
    nsl, d = h.shape
    hid = w1.shape[1]

    def body(h_ref, w1_ref, b1_ref, w2_ref, b2_ref, o_ref):
        z = jnp.dot(h_ref[...], w1_ref[...], preferred_element_type=F32)
        z = jnp.maximum(z + b1_ref[...], 0.0)
        o_ref[...] = jnp.dot(z, w2_ref[...],
                             preferred_element_type=F32) + b2_ref[...]

    return pl.pallas_call(
        body,
        grid=(nsl // nblk,),
        in_specs=[
            pl.BlockSpec((nblk, d), lambda i: (i, 0)),
            pl.BlockSpec((d, hid), lambda i: (0, 0)),
            pl.BlockSpec((1, hid), lambda i: (0, 0)),
            pl.BlockSpec((hid, 1), lambda i: (0, 0)),
            pl.BlockSpec((1, 1), lambda i: (0, 0)),
        ],
        out_specs=pl.BlockSpec((nblk, 1), lambda i: (i, 0)),
        out_shape=jax.ShapeDtypeStruct((nsl, 1), F32),
    )(h, w1, b1, w2, b2)


def kernel(edge_features, params, lg_edge_index):
    n, f_in = edge_features.shape
    e = lg_edge_index.shape[1]
    d = params['proj_W2'].shape[1]

    src = lg_edge_index[0]
    dst = lg_edge_index[1]

    ept = -(-e // (NS * GROUP)) * GROUP
    epad = ept * NS
    pad = epad - e
    nslab = (n // 512 + 1) * 512

    src2 = src * 2
    gi = jnp.stack([src2, src2 + 1])
    gir = jnp.pad(gi, ((0, 0), (0, pad)))
    dstp = jnp.pad(dst, (0, pad), constant_values=n)
    dstr = dstp
    zeros = jnp.zeros((nslab, LANES), F32)

    mesh = plsc.VectorSubcoreMesh(core_axis_name="c", subcore_axis_name="s",
                                  num_cores=NC, num_subcores=NS)

    degp = _sc_deg_call(dstp, nslab, epad, mesh)

    nblk = 2048
    xp = jnp.pad(edge_features, ((0, nslab - n), (0, 0)))
    p = params
    ht = _tc_proj(xp, p['proj_W1'], p['proj_b1'][None, :],
                  p['proj_W2'], p['proj_b2'][None, :], nblk)

    for blk in p['blocks']:
        h2 = ht.reshape(nslab * 2, LANES)
        aggs = _sc_layer_call(h2, gir, dstr, zeros, nslab, ept, mesh)
        w1 = blk['W1']
        ht = _tc_block(ht, aggs[0], aggs[1], degp,
                       w1[:d], w1[d:d + LANES], w1[d + LANES:],
                       blk['b1'][None, :], blk['W2'], blk['b2'][None, :],
                       blk['gamma'][None, :], blk['beta'][None, :],
                       nblk)

    ot = _tc_out(ht, p['out_W1'], p['out_b1'][None, :],
                 p['out_W2'], p['out_b2'][None, :], nblk)
    return ot[:n]


# speedup vs baseline: 1.0590x; 1.0590x over previous
import functools

import jax
import jax.numpy as jnp
from jax import lax
from jax.experimental import pallas as pl
from jax.experimental.pallas import tpu as pltpu
from jax.experimental.pallas import tpu_sc as plsc

F32 = jnp.float32

NC = 2
NS = 16
LANES = 16
GK = 4
GROUP = GK * 128
DGROUP = 1024


def _sc_deg_call(dstp, nslab, epad, mesh):
    nw = NC * NS
    epw = epad // nw

    def body(dst_ref, out_ref, didx, hist):
        c = lax.axis_index("c")
        s = lax.axis_index("s")
        w = s * NC + c
        z16 = jnp.zeros((LANES,), F32)

        def zb(i, _):
            hist[pl.ds(i * LANES, LANES)] = z16
            return 0

        lax.fori_loop(0, nslab // LANES, zb, 0)
        ones16 = jnp.ones((LANES,), F32)

        def gb(g, _):
            e0 = w * epw + g * DGROUP
            pltpu.sync_copy(dst_ref.at[pl.ds(e0, DGROUP)], didx)

            def ib(i, _):
                v = didx[pl.ds(i * LANES, LANES)]
                plsc.addupdate_scatter(hist, [v], ones16)
                return 0

            lax.fori_loop(0, DGROUP // LANES, ib, 0)
            return 0

        lax.fori_loop(0, epw // DGROUP, gb, 0)
        pltpu.sync_copy(hist, out_ref.at[w])

    f = pl.kernel(
        body,
        out_type=jax.ShapeDtypeStruct((nw, nslab), F32),
        mesh=mesh,
        scratch_types=[
            pltpu.VMEM((DGROUP,), jnp.int32),
            pltpu.VMEM((nslab,), F32),
        ],
        compiler_params=pltpu.CompilerParams(needs_layout_passes=False),
    )
    return f(dstp)


def _sc_layer_call(h2, gir, dstr, zeros, nslab, ept, mesh):
    rpt = ept // 128
    zpt = nslab // NS
    ngroups = ept // GROUP

    def body(h2_ref, gi_ref, dst_ref, z_ref, out_ref,
             gidx, didx, rows, slab, gsem, ssem, isem):
        c = lax.axis_index("c")
        s = lax.axis_index("s")
        pltpu.sync_copy(z_ref.at[pl.ds(s * zpt, zpt)],
                        slab.at[pl.ds(s * zpt, zpt)])
        plsc.subcore_barrier()

        def prefetch_idx(g, p):
            e0 = s * ept + g * GROUP
            pltpu.async_copy(gi_ref.at[c, pl.ds(e0, GROUP)], gidx.at[p], isem)
            pltpu.async_copy(dst_ref.at[pl.ds(e0, GROUP)], didx.at[p], isem)

        def wait_idx(g, p):
            e0 = s * ept + g * GROUP
            pltpu.make_async_copy(gi_ref.at[c, pl.ds(e0, GROUP)],
                                  gidx.at[p], isem).wait()
            pltpu.make_async_copy(dst_ref.at[pl.ds(e0, GROUP)],
                                  didx.at[p], isem).wait()

        def fire_gather(p):
            pltpu.async_copy(h2_ref.at[gidx.at[p]], rows.at[p], gsem)

        def wait_gather(p):
            pltpu.make_async_copy(h2_ref.at[gidx.at[p]], rows.at[p],
                                  gsem).wait()

        def fire_scatter(p):
            pltpu.async_copy(rows.at[p], slab.at[didx.at[p]], ssem, add=True)

        def drain_scatter(p):
            pltpu.make_async_copy(rows.at[p], slab.at[didx.at[p]],
                                  ssem).wait()

        prefetch_idx(0, 0)
        prefetch_idx(1, 1)
        wait_idx(0, 0)
        fire_gather(0)

        def gb(g, _):
            r = lax.rem(g, 3)
            r1 = lax.rem(g + 1, 3)
            r2 = lax.rem(g + 2, 3)

            @pl.when(g < ngroups - 1)
            def _():
                wait_idx(g + 1, r1)
                fire_gather(r1)

            @pl.when(g >= 1)
            def _():
                drain_scatter(r2)

            @pl.when(g < ngroups - 2)
            def _():
                prefetch_idx(g + 2, r2)

            wait_gather(r)
            fire_scatter(r)
            return 0

        lax.fori_loop(0, ngroups, gb, 0)
        drain_scatter(lax.rem(ngroups - 1, 3))
        plsc.subcore_barrier()
        pltpu.sync_copy(slab.at[pl.ds(s * zpt, zpt)],
                        out_ref.at[c, pl.ds(s * zpt, zpt)])

    f = pl.kernel(
        body,
        out_type=jax.ShapeDtypeStruct((NC, nslab, LANES), F32),
        mesh=mesh,
        scratch_types=[
            pltpu.VMEM((3, GROUP), jnp.int32),
            pltpu.VMEM((3, GROUP), jnp.int32),
            pltpu.VMEM((3, GROUP, LANES), F32),
            pltpu.VMEM_SHARED((nslab, LANES), F32),
            pltpu.SemaphoreType.DMA,
            pltpu.SemaphoreType.DMA,
            pltpu.SemaphoreType.DMA,
        ],
        compiler_params=pltpu.CompilerParams(use_tc_tiling_on_sc=False),
    )
    return f(h2, gir, dstr, zeros)


def _tc_proj(xt, w1t, b1t, w2t, b2t, lanes_blk):
    f_in, nsl = xt.shape
    d = w2t.shape[0]
    hid = w1t.shape[0]

    def body(x_ref, w1_ref, b1_ref, w2_ref, b2_ref, h_ref):
        z = jnp.dot(w1_ref[...], x_ref[...], preferred_element_type=F32)
        z = jnp.maximum(z + b1_ref[...], 0.0)
        h_ref[...] = jnp.dot(w2_ref[...], z,
                             preferred_element_type=F32) + b2_ref[...]

    return pl.pallas_call(
        body,
        grid=(nsl // lanes_blk,),
        in_specs=[
            pl.BlockSpec((f_in, lanes_blk), lambda i: (0, i)),
            pl.BlockSpec((hid, f_in), lambda i: (0, 0)),
            pl.BlockSpec((hid, 1), lambda i: (0, 0)),
            pl.BlockSpec((d, hid), lambda i: (0, 0)),
            pl.BlockSpec((d, 1), lambda i: (0, 0)),
        ],
        out_specs=pl.BlockSpec((d, lanes_blk), lambda i: (0, i)),
        out_shape=jax.ShapeDtypeStruct((d, nsl), F32),
    )(xt, w1t, b1t, w2t, b2t)


def _tc_block(ht, a0t, a1t, degp, w1ht, w1at, w1bt, b1t, w2t, b2t,
              gam, bet, lanes_blk):
    d, nsl = ht.shape
    hid = w2t.shape[1]
    nw = degp.shape[0]

    def body(h_ref, a0_ref, a1_ref, degp_ref, w1h_ref, w1a_ref, w1b_ref,
             b1_ref, w2_ref, b2_ref, gam_ref, bet_ref, o_ref):
        h_ = h_ref[...]
        deg = jnp.sum(degp_ref[...], axis=0, keepdims=True)
        di = 1.0 / jnp.maximum(deg, 1.0)
        z = (jnp.dot(w1h_ref[...], h_, preferred_element_type=F32)
             + jnp.dot(w1a_ref[...], a0_ref[...] * di,
                       preferred_element_type=F32)
             + jnp.dot(w1b_ref[...], a1_ref[...] * di,
                       preferred_element_type=F32)
             + b1_ref[...])
        z = jnp.maximum(z, 0.0)
        hn = jnp.dot(w2_ref[...], z, preferred_element_type=F32) + b2_ref[...]
        r = h_ + hn
        mu = jnp.mean(r, axis=0, keepdims=True)
        var = jnp.mean((r - mu) ** 2, axis=0, keepdims=True)
        o_ref[...] = ((r - mu) / jnp.sqrt(var + 1e-5) * gam_ref[...]
                      + bet_ref[...])

    return pl.pallas_call(
        body,
        grid=(nsl // lanes_blk,),
        in_specs=[
            pl.BlockSpec((d, lanes_blk), lambda i: (0, i)),
            pl.BlockSpec((LANES, lanes_blk), lambda i: (0, i)),
            pl.BlockSpec((LANES, lanes_blk), lambda i: (0, i)),
            pl.BlockSpec((nw, lanes_blk), lambda i: (0, i)),
            pl.BlockSpec((hid, d), lambda i: (0, 0)),
            pl.BlockSpec((hid, LANES), lambda i: (0, 0)),
            pl.BlockSpec((hid, LANES), lambda i: (0, 0)),
            pl.BlockSpec((hid, 1), lambda i: (0, 0)),
            pl.BlockSpec((d, hid), lambda i: (0, 0)),
            pl.BlockSpec((d, 1), lambda i: (0, 0)),
            pl.BlockSpec((d, 1), lambda i: (0, 0)),
            pl.BlockSpec((d, 1), lambda i: (0, 0)),
        ],
        out_specs=pl.BlockSpec((d, lanes_blk), lambda i: (0, i)),
        out_shape=jax.ShapeDtypeStruct((d, nsl), F32),
    )(ht, a0t, a1t, degp, w1ht, w1at, w1bt, b1t, w2t, b2t, gam, bet)


def _tc_out(ht, w1t, b1t, w2t, b2t, lanes_blk):
    d, nsl = ht.shape
    hid = w1t.shape[0]

    def body(h_ref, w1_ref, b1_ref, w2_ref, b2_ref, o_ref):
        z = jnp.dot(w1_ref[...], h_ref[...], preferred_element_type=F32)
        z = jnp.maximum(z + b1_ref[...], 0.0)
        o_ref[...] = jnp.dot(w2_ref[...], z,
                             preferred_element_type=F32) + b2_ref[...]

    return pl.pallas_call(
        body,
        grid=(nsl // lanes_blk,),
        in_specs=[
            pl.BlockSpec((d, lanes_blk), lambda i: (0, i)),
            pl.BlockSpec((hid, d), lambda i: (0, 0)),
            pl.BlockSpec((hid, 1), lambda i: (0, 0)),
            pl.BlockSpec((1, hid), lambda i: (0, 0)),
            pl.BlockSpec((1, 1), lambda i: (0, 0)),
        ],
        out_specs=pl.BlockSpec((1, lanes_blk), lambda i: (0, i)),
        out_shape=jax.ShapeDtypeStruct((1, nsl), F32),
    )(ht, w1t, b1t, w2t, b2t)


def kernel(edge_features, params, lg_edge_index):
    n, f_in = edge_features.shape
    e = lg_edge_index.shape[1]
    d = params['proj_W2'].shape[1]

    src = lg_edge_index[0]
    dst = lg_edge_index[1]

    ept = -(-e // (NS * GROUP)) * GROUP
    epad = ept * NS
    pad = epad - e
    nslab = (n // 512 + 1) * 512

    src2 = src * 2
    gi = jnp.stack([src2, src2 + 1])
    gir = jnp.pad(gi, ((0, 0), (0, pad)))
    dstp = jnp.pad(dst, (0, pad), constant_values=n)
    dstr = dstp
    zeros = jnp.zeros((nslab, LANES), F32)

    mesh = plsc.VectorSubcoreMesh(core_axis_name="c", subcore_axis_name="s",
                                  num_cores=NC, num_subcores=NS)

    degp = _sc_deg_call(dstp, nslab, epad, mesh)

    lanes_blk = 2048
    xt = jnp.pad(edge_features.T, ((0, 0), (0, nslab - n)))
    p = params
    ht = _tc_proj(xt, p['proj_W1'].T, p['proj_b1'][:, None],
                  p['proj_W2'].T, p['proj_b2'][:, None], lanes_blk)

    for blk in p['blocks']:
        h2 = ht.T.reshape(nslab * 2, LANES)
        aggs = _sc_layer_call(h2, gir, dstr, zeros, nslab, ept, mesh)
        w1 = blk['W1']
        ht = _tc_block(ht, aggs[0].T, aggs[1].T, degp,
                       w1[:d].T, w1[d:d + LANES].T, w1[d + LANES:].T,
                       blk['b1'][:, None], blk['W2'].T, blk['b2'][:, None],
                       blk['gamma'][:, None], blk['beta'][:, None],
                       lanes_blk)

    ot = _tc_out(ht, p['out_W1'].T, p['out_b1'][:, None],
                 p['out_W2'].T, p['out_b2'][:, None], lanes_blk)
    return ot[0, :n, None]
